# Initial kernel scaffold; baseline (speedup 1.0000x reference)
import jax
import jax.numpy as jnp
from jax.experimental import pallas as pl


def kernel(x, edge_index, batch, eps0, W1_0, b1_0, W2_0, b2_0, eps1, W1_1, b1_1, W2_1, b2_1, eps2, W1_2, b1_2, W2_2, b2_2):
    raise NotImplementedError("write your pallas kernel here")


# speedup vs baseline: 2.7372x; 2.7372x over previous
import functools

import jax
import jax.numpy as jnp
from jax import lax
from jax.experimental import pallas as pl
from jax.experimental.pallas import tpu as pltpu
from jax.experimental.pallas import tpu_sc as plsc

N = 10000
D = 128
H_DIM = 256
E = 320000

NC, NS, L = 2, 16, 16
NW = NC * NS
NPAD = 10112
RPT = NPAD // NS
CHUNK = 128
CHUNKS = 80
EPAD = NW * CHUNKS * CHUNK


def _sc_aggregate(h, srcp, dstp, zrows):
    mesh = plsc.VectorSubcoreMesh(
        core_axis_name="c", subcore_axis_name="s", num_cores=NC, num_subcores=NS
    )

    @functools.partial(
        pl.kernel,
        mesh=mesh,
        out_type=jax.ShapeDtypeStruct((NC, NPAD, D), jnp.float32),
        scratch_types=[
            pltpu.VMEM((CHUNKS, CHUNK), jnp.int32),
            pltpu.VMEM((CHUNKS, CHUNK), jnp.int32),
            pltpu.VMEM((CHUNK, D), jnp.float32),
            pltpu.VMEM_SHARED((NPAD, D), jnp.float32),
            pltpu.SemaphoreType.DMA,
        ],
    )
    def body(h_hbm, src_hbm, dst_hbm, z_hbm, out_hbm, src_v, dst_v, rows_v, agg_sh, sem):
        cid = lax.axis_index("c")
        sid = lax.axis_index("s")
        wid = cid * NS + sid
        pltpu.sync_copy(z_hbm, agg_sh.at[pl.ds(sid * RPT, RPT)])
        pltpu.sync_copy(src_hbm.at[pl.ds(wid * CHUNKS, CHUNKS)], src_v)
        pltpu.sync_copy(dst_hbm.at[pl.ds(wid * CHUNKS, CHUNKS)], dst_v)
        plsc.subcore_barrier()

        @pl.loop(0, CHUNKS)
        def _(j):
            pltpu.async_copy(h_hbm.at[src_v.at[j]], rows_v, sem).wait()
            pltpu.sync_copy(rows_v, agg_sh.at[dst_v.at[j]], add=True)

        plsc.subcore_barrier()
        pltpu.sync_copy(
            agg_sh.at[pl.ds(sid * RPT, RPT)],
            out_hbm.at[cid].at[pl.ds(sid * RPT, RPT)],
        )

    return body(h, srcp, dstp, zrows)


def _mlp_block(scale_ref, h_ref, a0_ref, a1_ref, w1_ref, b1_ref, w2_ref, b2_ref, o_ref):
    z = scale_ref[0] * h_ref[...] + a0_ref[...] + a1_ref[...]
    z = jnp.maximum(
        jnp.dot(z, w1_ref[...], preferred_element_type=jnp.float32) + b1_ref[...], 0.0
    )
    z = jnp.dot(z, w2_ref[...], preferred_element_type=jnp.float32) + b2_ref[...]
    o_ref[...] = jnp.maximum(z, 0.0)


def _tc_mlp(h, a0, a1, scale, W1, b1, W2, b2):
    R = 1000
    grid = (N // R,)
    return pl.pallas_call(
        _mlp_block,
        grid=grid,
        in_specs=[
            pl.BlockSpec(memory_space=pltpu.SMEM),
            pl.BlockSpec((R, D), lambda i: (i, 0)),
            pl.BlockSpec((R, D), lambda i: (i, 0)),
            pl.BlockSpec((R, D), lambda i: (i, 0)),
            pl.BlockSpec((D, H_DIM), lambda i: (0, 0)),
            pl.BlockSpec((1, H_DIM), lambda i: (0, 0)),
            pl.BlockSpec((H_DIM, D), lambda i: (0, 0)),
            pl.BlockSpec((1, D), lambda i: (0, 0)),
        ],
        out_specs=pl.BlockSpec((R, D), lambda i: (i, 0)),
        out_shape=jax.ShapeDtypeStruct((N, D), jnp.float32),
    )(scale, h, a0, a1, W1, b1.reshape(1, H_DIM), W2, b2.reshape(1, D))


def kernel(x, edge_index, batch,
           eps0, W1_0, b1_0, W2_0, b2_0,
           eps1, W1_1, b1_1, W2_1, b2_1,
           eps2, W1_2, b1_2, W2_2, b2_2):
    src = edge_index[0]
    dst = edge_index[1]
    pad = EPAD - E
    srcp = jnp.concatenate([src, jnp.zeros((pad,), jnp.int32)]).reshape(EPAD // CHUNK, CHUNK)
    dstp = jnp.concatenate([dst, jnp.full((pad,), N, jnp.int32)]).reshape(EPAD // CHUNK, CHUNK)
    zrows = jnp.zeros((RPT, D), jnp.float32)

    h = x
    for (eps, W1, b1, W2, b2) in (
        (eps0, W1_0, b1_0, W2_0, b2_0),
        (eps1, W1_1, b1_1, W2_1, b2_1),
        (eps2, W1_2, b1_2, W2_2, b2_2),
    ):
        agg = _sc_aggregate(h, srcp, dstp, zrows)
        scale = jnp.reshape(1.0 + eps, (1,)).astype(jnp.float32)
        h = _tc_mlp(h, agg[0, :N], agg[1, :N], scale, W1, b1, W2, b2)
    return h


# speedup vs baseline: 3.2182x; 1.1758x over previous
import functools

import jax
import jax.numpy as jnp
from jax import lax
from jax.experimental import pallas as pl
from jax.experimental.pallas import tpu as pltpu
from jax.experimental.pallas import tpu_sc as plsc

N = 10000
D = 128
H_DIM = 256
E = 320000

NC, NS, L = 2, 16, 16
NW = NC * NS
NPAD = 10112
RPT = NPAD // NS
CHUNK = 64
CHUNKS = 160
EPAD = NW * CHUNKS * CHUNK
NR = 4
A = 2
S = 2
P = 2 * NR
U = P - S
SRC_BITS = 14
SRC_MASK = (1 << SRC_BITS) - 1
PROWS = CHUNKS * CHUNK // 128
CPR = 128 // CHUNK


def _sc_aggregate(h, packed, zrows):
    mesh = plsc.VectorSubcoreMesh(
        core_axis_name="c", subcore_axis_name="s", num_cores=NC, num_subcores=NS
    )

    @functools.partial(
        pl.kernel,
        mesh=mesh,
        out_type=jax.ShapeDtypeStruct((NC, NPAD, D), jnp.float32),
        scratch_types=[
            pltpu.VMEM((PROWS, 128), jnp.int32),
            pltpu.VMEM((P, CHUNK), jnp.int32),
            pltpu.VMEM((P, CHUNK), jnp.int32),
            pltpu.VMEM((NR, CHUNK, D), jnp.float32),
            pltpu.VMEM_SHARED((NPAD, D), jnp.float32),
            pltpu.SemaphoreType.DMA,
            pltpu.SemaphoreType.DMA,
        ],
    )
    def body(h_hbm, pk_hbm, z_hbm, out_hbm, pk_v, sidx_v, didx_v, rows_v, agg_sh,
             gsem, ssem):
        cid = lax.axis_index("c")
        sid = lax.axis_index("s")
        wid = cid * NS + sid
        pltpu.sync_copy(z_hbm, agg_sh.at[pl.ds(sid * RPT, RPT)])
        pltpu.sync_copy(pk_hbm.at[pl.ds(wid * PROWS, PROWS)], pk_v)
        plsc.subcore_barrier()

        def unpack(c, k):
            row = c // CPR
            base = (k % CPR) * CHUNK
            slot = k % P
            for v in range(CHUNK // L):
                p = pk_v[row, pl.ds(base + v * L, L)]
                sidx_v[slot, pl.ds(v * L, L)] = jnp.bitwise_and(p, SRC_MASK)
                didx_v[slot, pl.ds(v * L, L)] = lax.shift_right_logical(p, SRC_BITS)

        def step(c, k, wait_scat, do_unpack, do_gather):
            b = k % NR
            pltpu.make_async_copy(h_hbm.at[sidx_v.at[k % P]], rows_v.at[b], gsem).wait()
            pltpu.async_copy(rows_v.at[b], agg_sh.at[didx_v.at[k % P]], ssem, add=True)
            if wait_scat:
                pltpu.make_async_copy(
                    rows_v.at[(k - S) % NR], agg_sh.at[didx_v.at[(k - S) % P]], ssem
                ).wait()
            if do_unpack:
                unpack(c + U, k + U)
            if do_gather:
                pltpu.async_copy(
                    h_hbm.at[sidx_v.at[(k + A) % P]], rows_v.at[(k + A) % NR], gsem
                )

        for c in range(U):
            unpack(c, c)
        for b in range(A):
            pltpu.async_copy(h_hbm.at[sidx_v.at[b]], rows_v.at[b], gsem)

        for k in range(P):
            step(k, k, k >= S, True, True)

        @pl.loop(P, CHUNKS - P, step=P)
        def _(g):
            for k in range(P):
                step(g + k, k, True, True, True)

        base_c = CHUNKS - P
        for k in range(P):
            step(base_c + k, k, True, k < S, k < P - A)
        for k in range(P - S, P):
            pltpu.make_async_copy(
                rows_v.at[k % NR], agg_sh.at[didx_v.at[k]], ssem
            ).wait()

        plsc.subcore_barrier()
        pltpu.sync_copy(
            agg_sh.at[pl.ds(sid * RPT, RPT)],
            out_hbm.at[cid].at[pl.ds(sid * RPT, RPT)],
        )

    return body(h, packed, zrows)


def _mlp_block(scale_ref, h_ref, a0_ref, a1_ref, w1_ref, b1_ref, w2_ref, b2_ref, o_ref):
    z = scale_ref[0] * h_ref[...] + a0_ref[...] + a1_ref[...]
    z = jnp.maximum(
        jnp.dot(z, w1_ref[...], preferred_element_type=jnp.float32) + b1_ref[...], 0.0
    )
    z = jnp.dot(z, w2_ref[...], preferred_element_type=jnp.float32) + b2_ref[...]
    o_ref[...] = jnp.maximum(z, 0.0)


def _tc_mlp(h, a0, a1, scale, W1, b1, W2, b2):
    R = 1000
    grid = (N // R,)
    return pl.pallas_call(
        _mlp_block,
        grid=grid,
        in_specs=[
            pl.BlockSpec(memory_space=pltpu.SMEM),
            pl.BlockSpec((R, D), lambda i: (i, 0)),
            pl.BlockSpec((R, D), lambda i: (i, 0)),
            pl.BlockSpec((R, D), lambda i: (i, 0)),
            pl.BlockSpec((D, H_DIM), lambda i: (0, 0)),
            pl.BlockSpec((1, H_DIM), lambda i: (0, 0)),
            pl.BlockSpec((H_DIM, D), lambda i: (0, 0)),
            pl.BlockSpec((1, D), lambda i: (0, 0)),
        ],
        out_specs=pl.BlockSpec((R, D), lambda i: (i, 0)),
        out_shape=jax.ShapeDtypeStruct((N, D), jnp.float32),
    )(scale, h, a0, a1, W1, b1.reshape(1, H_DIM), W2, b2.reshape(1, D))


def kernel(x, edge_index, batch,
           eps0, W1_0, b1_0, W2_0, b2_0,
           eps1, W1_1, b1_1, W2_1, b2_1,
           eps2, W1_2, b1_2, W2_2, b2_2):
    src = edge_index[0]
    dst = edge_index[1]
    pad = EPAD - E
    srcp = jnp.concatenate([src, jnp.zeros((pad,), jnp.int32)])
    dstp = jnp.concatenate([dst, jnp.full((pad,), N, jnp.int32)])
    packed = (lax.shift_left(dstp, SRC_BITS) | srcp).reshape(EPAD // 128, 128)
    zrows = jnp.zeros((RPT, D), jnp.float32)

    h = x
    for (eps, W1, b1, W2, b2) in (
        (eps0, W1_0, b1_0, W2_0, b2_0),
        (eps1, W1_1, b1_1, W2_1, b2_1),
        (eps2, W1_2, b1_2, W2_2, b2_2),
    ):
        agg = _sc_aggregate(h, packed, zrows)
        scale = jnp.reshape(1.0 + eps, (1,)).astype(jnp.float32)
        h = _tc_mlp(h, agg[0, :N], agg[1, :N], scale, W1, b1, W2, b2)
    return h


# speedup vs baseline: 3.2276x; 1.0029x over previous
import functools

import jax
import jax.numpy as jnp
from jax import lax
from jax.experimental import pallas as pl
from jax.experimental.pallas import tpu as pltpu
from jax.experimental.pallas import tpu_sc as plsc

N = 10000
D = 128
H_DIM = 256
E = 320000

NC, NS, L = 2, 16, 16
NW = NC * NS
NPAD = 10112
RPT = NPAD // NS
CHUNK = 64
CHUNKS = 160
EPAD = NW * CHUNKS * CHUNK
NR = 4
A = 3
S = 1
P = 2 * NR
U = P - S
SRC_BITS = 14
SRC_MASK = (1 << SRC_BITS) - 1
PROWS = CHUNKS * CHUNK // 128
CPR = 128 // CHUNK


def _sc_aggregate(h, packed, zrows):
    mesh = plsc.VectorSubcoreMesh(
        core_axis_name="c", subcore_axis_name="s", num_cores=NC, num_subcores=NS
    )

    @functools.partial(
        pl.kernel,
        mesh=mesh,
        out_type=jax.ShapeDtypeStruct((NC, NPAD, D), jnp.float32),
        scratch_types=[
            pltpu.VMEM((PROWS, 128), jnp.int32),
            pltpu.VMEM((P, CHUNK), jnp.int32),
            pltpu.VMEM((P, CHUNK), jnp.int32),
            pltpu.VMEM((NR, CHUNK, D), jnp.float32),
            pltpu.VMEM_SHARED((NPAD, D), jnp.float32),
            pltpu.SemaphoreType.DMA,
            pltpu.SemaphoreType.DMA,
        ],
    )
    def body(h_hbm, pk_hbm, z_hbm, out_hbm, pk_v, sidx_v, didx_v, rows_v, agg_sh,
             gsem, ssem):
        cid = lax.axis_index("c")
        sid = lax.axis_index("s")
        wid = cid * NS + sid
        pltpu.sync_copy(z_hbm, agg_sh.at[pl.ds(sid * RPT, RPT)])
        pltpu.sync_copy(pk_hbm.at[pl.ds(wid * PROWS, PROWS)], pk_v)
        plsc.subcore_barrier()

        def unpack(c, k):
            row = c // CPR
            base = (k % CPR) * CHUNK
            slot = k % P
            for v in range(CHUNK // L):
                p = pk_v[row, pl.ds(base + v * L, L)]
                sidx_v[slot, pl.ds(v * L, L)] = jnp.bitwise_and(p, SRC_MASK)
                didx_v[slot, pl.ds(v * L, L)] = lax.shift_right_logical(p, SRC_BITS)

        def step(c, k, wait_scat, do_unpack, do_gather):
            b = k % NR
            pltpu.make_async_copy(h_hbm.at[sidx_v.at[k % P]], rows_v.at[b], gsem).wait()
            pltpu.async_copy(rows_v.at[b], agg_sh.at[didx_v.at[k % P]], ssem, add=True)
            if wait_scat:
                pltpu.make_async_copy(
                    rows_v.at[(k - S) % NR], agg_sh.at[didx_v.at[(k - S) % P]], ssem
                ).wait()
            if do_unpack:
                unpack(c + U, k + U)
            if do_gather:
                pltpu.async_copy(
                    h_hbm.at[sidx_v.at[(k + A) % P]], rows_v.at[(k + A) % NR], gsem
                )

        for c in range(U):
            unpack(c, c)
        for b in range(A):
            pltpu.async_copy(h_hbm.at[sidx_v.at[b]], rows_v.at[b], gsem)

        for k in range(P):
            step(k, k, k >= S, True, True)

        @pl.loop(P, CHUNKS - P, step=P)
        def _(g):
            for k in range(P):
                step(g + k, k, True, True, True)

        base_c = CHUNKS - P
        for k in range(P):
            step(base_c + k, k, True, k < S, k < P - A)
        for k in range(P - S, P):
            pltpu.make_async_copy(
                rows_v.at[k % NR], agg_sh.at[didx_v.at[k]], ssem
            ).wait()

        plsc.subcore_barrier()
        pltpu.sync_copy(
            agg_sh.at[pl.ds(sid * RPT, RPT)],
            out_hbm.at[cid].at[pl.ds(sid * RPT, RPT)],
        )

    return body(h, packed, zrows)


def _mlp_block(scale_ref, h_ref, a0_ref, a1_ref, w1_ref, b1_ref, w2_ref, b2_ref, o_ref):
    z = scale_ref[0] * h_ref[...] + a0_ref[...] + a1_ref[...]
    z = jnp.maximum(
        jnp.dot(z, w1_ref[...], preferred_element_type=jnp.float32) + b1_ref[...], 0.0
    )
    z = jnp.dot(z, w2_ref[...], preferred_element_type=jnp.float32) + b2_ref[...]
    o_ref[...] = jnp.maximum(z, 0.0)


def _tc_mlp(h, a0, a1, scale, W1, b1, W2, b2):
    R = 1000
    grid = (N // R,)
    return pl.pallas_call(
        _mlp_block,
        grid=grid,
        in_specs=[
            pl.BlockSpec(memory_space=pltpu.SMEM),
            pl.BlockSpec((R, D), lambda i: (i, 0)),
            pl.BlockSpec((R, D), lambda i: (i, 0)),
            pl.BlockSpec((R, D), lambda i: (i, 0)),
            pl.BlockSpec((D, H_DIM), lambda i: (0, 0)),
            pl.BlockSpec((1, H_DIM), lambda i: (0, 0)),
            pl.BlockSpec((H_DIM, D), lambda i: (0, 0)),
            pl.BlockSpec((1, D), lambda i: (0, 0)),
        ],
        out_specs=pl.BlockSpec((R, D), lambda i: (i, 0)),
        out_shape=jax.ShapeDtypeStruct((N, D), jnp.float32),
    )(scale, h, a0, a1, W1, b1.reshape(1, H_DIM), W2, b2.reshape(1, D))


def kernel(x, edge_index, batch,
           eps0, W1_0, b1_0, W2_0, b2_0,
           eps1, W1_1, b1_1, W2_1, b2_1,
           eps2, W1_2, b1_2, W2_2, b2_2):
    src = edge_index[0]
    dst = edge_index[1]
    pad = EPAD - E
    srcp = jnp.concatenate([src, jnp.zeros((pad,), jnp.int32)])
    dstp = jnp.concatenate([dst, jnp.full((pad,), N, jnp.int32)])
    packed = (lax.shift_left(dstp, SRC_BITS) | srcp).reshape(EPAD // 128, 128)
    zrows = jnp.zeros((RPT, D), jnp.float32)

    h = x
    for (eps, W1, b1, W2, b2) in (
        (eps0, W1_0, b1_0, W2_0, b2_0),
        (eps1, W1_1, b1_1, W2_1, b2_1),
        (eps2, W1_2, b1_2, W2_2, b2_2),
    ):
        agg = _sc_aggregate(h, packed, zrows)
        scale = jnp.reshape(1.0 + eps, (1,)).astype(jnp.float32)
        h = _tc_mlp(h, agg[0, :N], agg[1, :N], scale, W1, b1, W2, b2)
    return h
